# Initial kernel scaffold; baseline (speedup 1.0000x reference)
import jax
import jax.numpy as jnp
from jax.experimental import pallas as pl


def kernel(pseudo_boxes, gt_bboxes, merge_boxes):
    raise NotImplementedError("write your pallas kernel here")


# speedup vs baseline: 1.4758x; 1.4758x over previous
import functools

import numpy as np
import jax
import jax.numpy as jnp
from jax import lax
from jax.experimental import pallas as pl
from jax.experimental.pallas import tpu as pltpu
from jax.experimental.pallas import tpu_sc as plsc

N_ROWS = 50000
N_BOX = 64
NC, NS, L = 2, 16, 16
NW = NC * NS
GROUPS = N_ROWS // L
CG = 5
CHUNKS = GROUPS // CG
BIG_W = CHUNKS - 19 * NW
GROUP_F32 = L * N_BOX * 4
CHUNK_F32 = CG * GROUP_F32
GT_CH_F32 = CG * L * 4

B1, B2, B3, B4, B5, B6, B7, B8, B9 = 0, 40, 80, 100, 120, 140, 160, 180, 220
SLOTS = 272
EPS = np.float32(1e-6)
HALF = np.float32(0.5)
F0 = np.float32(0.0)
F1 = np.float32(1.0)

INV40 = np.float32(1.0) / np.float32(2.0 / 40.0)
INV20 = np.float32(1.0) / np.float32(1.0 / 20.0)
INV20W = np.float32(1.0) / np.float32(2.0 / 20.0)
F40 = np.float32(40.0)
F20 = np.float32(20.0)


def _sc_body(pb_hbm, gt_hbm, mb_hbm, out_hbm, pb_v, gt_v, mb_v, part_v, hist):
  wid = lax.axis_index("s") * NC + lax.axis_index("c")
  lane = lax.iota(jnp.int32, L)
  lane4 = lane * 4
  lane256 = lane * 256
  zeros = jnp.zeros((L,), jnp.float32)
  ones = jnp.ones((L,), jnp.float32)

  lh1 = lane + B1 * L
  lh2 = lane + B2 * L
  lh3 = lane + B3 * L
  D8 = (B8 - B1) * L
  D9 = (B9 - B2) * L

  def zero_slot(s, _):
    hist[pl.ds(s * L, L)] = zeros
    return 0

  lax.fori_loop(0, SLOTS, zero_slot, 0)

  chunk0 = wid * 19 + jnp.minimum(wid, BIG_W)
  n_chunks = jnp.where(wid < BIG_W, 20, 19)

  def chunk_body(s, _):
    c = chunk0 + s
    pltpu.sync_copy(pb_hbm.at[pl.ds(c * CHUNK_F32, CHUNK_F32)], pb_v)
    pltpu.sync_copy(gt_hbm.at[pl.ds(c * GT_CH_F32, GT_CH_F32)], gt_v)
    pltpu.sync_copy(mb_hbm.at[pl.ds(c * GT_CH_F32, GT_CH_F32)], mb_v)

    for q in range(CG):
      qg = q * (L * 4)
      gx1 = plsc.load_gather(gt_v, [lane4 + qg])
      gy1 = plsc.load_gather(gt_v, [lane4 + (qg + 1)])
      gx2 = plsc.load_gather(gt_v, [lane4 + (qg + 2)])
      gy2 = plsc.load_gather(gt_v, [lane4 + (qg + 3)])
      mx1 = plsc.load_gather(mb_v, [lane4 + qg])
      my1 = plsc.load_gather(mb_v, [lane4 + (qg + 1)])
      mx2 = plsc.load_gather(mb_v, [lane4 + (qg + 2)])
      my2 = plsc.load_gather(mb_v, [lane4 + (qg + 3)])

      area_g = (gx2 - gx1) * (gy2 - gy1)
      area_m = (mx2 - mx1) * (my2 - my1)
      ww = jnp.maximum(jnp.minimum(gx2, mx2) - jnp.maximum(gx1, mx1), F0)
      hh = jnp.maximum(jnp.minimum(gy2, my2) - jnp.maximum(gy1, my1), F0)
      ov = ww * hh
      giou = ov / jnp.maximum(area_g + area_m - ov, EPS)
      recip_g = F1 / giou
      rn = recip_g * INV40
      a_m = (F1 - giou) * INV40
      g_big = giou >= HALF

      pbase = lane256 + q * GROUP_F32

      def box_body(j, run_max):
        jc = j * 4
        px1 = plsc.load_gather(pb_v, [pbase + jc])
        py1 = plsc.load_gather(pb_v, [pbase + (jc + 1)])
        px2 = plsc.load_gather(pb_v, [pbase + (jc + 2)])
        py2 = plsc.load_gather(pb_v, [pbase + (jc + 3)])
        area_p = (px2 - px1) * (py2 - py1)
        iw = jnp.maximum(jnp.minimum(px2, gx2) - jnp.maximum(px1, gx1), F0)
        ih = jnp.maximum(jnp.minimum(py2, gy2) - jnp.maximum(py1, gy1), F0)
        iov = iw * ih
        iou1 = iov / jnp.maximum(area_p + area_g - iov, EPS)

        t_n = iou1 * rn
        t_m = iou1 * INV40 + a_m
        t_1 = iou1 * INV20
        i_n = jnp.clip(t_n.astype(jnp.int32), 0, 39)
        i_m = jnp.clip(t_m.astype(jnp.int32), 0, 39)
        i_1 = jnp.clip(t_1.astype(jnp.int32), 0, 19)
        v_n = t_n <= F40
        v_m = (t_m >= F0) & (t_m <= F40)
        v_1 = t_1 <= F20
        nb = (iou1 >= HALF) | g_big

        a_n = i_n * L + lh1
        a_m2 = i_m * L + lh2
        a_1 = i_1 * L + lh3
        plsc.addupdate_scatter(hist, [a_n], ones, mask=v_n)
        plsc.addupdate_scatter(hist, [a_m2], ones, mask=v_m)
        plsc.addupdate_scatter(hist, [a_1], ones, mask=v_1)
        plsc.addupdate_scatter(hist, [a_n + D8], ones, mask=v_n & nb)
        plsc.addupdate_scatter(hist, [a_m2 + D9], ones, mask=v_m & nb)
        return jnp.maximum(run_max, iou1)

      max_iou1 = lax.fori_loop(0, N_BOX, box_body,
                               jnp.full((L,), -jnp.inf, jnp.float32))

      t4 = giou * INV20
      i4 = jnp.clip(t4.astype(jnp.int32), 0, 19)
      v4 = t4 <= F20
      t5 = max_iou1 * INV20
      i5 = jnp.clip(t5.astype(jnp.int32), 0, 19)
      v5 = (t5 >= F0) & (t5 <= F20)
      t7 = (giou + F1) * INV20W
      i7 = jnp.clip(t7.astype(jnp.int32), 0, 19)
      v7 = t7 <= F20
      plsc.addupdate_scatter(hist, [i4 * L + (lane + B4 * L)],
                             jnp.full((L,), 64.0, jnp.float32), mask=v4)
      plsc.addupdate_scatter(hist, [i5 * L + (lane + B5 * L)], ones, mask=v5)
      plsc.addupdate_scatter(hist, [i4 * L + (lane + B6 * L)], ones, mask=v4)
      plsc.addupdate_scatter(hist, [i7 * L + (lane + B7 * L)], ones, mask=v7)
    return 0

  lax.fori_loop(0, n_chunks, chunk_body, 0)

  for cc in range(SLOTS // L):
    base = (cc * L + lane) * L
    acc = zeros
    for l in range(L):
      acc = acc + plsc.load_gather(hist, [base + l])
    part_v[0, pl.ds(cc * L, L)] = acc

  pltpu.sync_copy(part_v, out_hbm.at[pl.ds(wid, 1)])


@jax.jit
def kernel(pseudo_boxes, gt_bboxes, merge_boxes):
  pb = pseudo_boxes.reshape(-1)
  gt = gt_bboxes.reshape(-1)
  mb = merge_boxes.reshape(-1)
  mesh = plsc.VectorSubcoreMesh(core_axis_name="c", subcore_axis_name="s")
  run = pl.kernel(
      _sc_body,
      out_type=jax.ShapeDtypeStruct((NW, SLOTS), jnp.float32),
      mesh=mesh,
      compiler_params=pltpu.CompilerParams(needs_layout_passes=False),
      scratch_types=[
          pltpu.VMEM((CHUNK_F32,), jnp.float32),
          pltpu.VMEM((GT_CH_F32,), jnp.float32),
          pltpu.VMEM((GT_CH_F32,), jnp.float32),
          pltpu.VMEM((1, SLOTS), jnp.float32),
          pltpu.VMEM((SLOTS * L,), jnp.float32),
      ],
  )
  partials = run(pb, gt, mb)
  return jnp.sum(partials, axis=0)[:260]


# speedup vs baseline: 88.8670x; 60.2171x over previous
import functools

import numpy as np
import jax
import jax.numpy as jnp
from jax import lax
from jax.experimental import pallas as pl
from jax.experimental.pallas import tpu as pltpu
from jax.experimental.pallas import tpu_sc as plsc

N_ROWS = 50000
N_BOX = 64
NC, NS, L = 2, 16, 16
NW = NC * NS
RB = 128
FULL_BLOCKS = N_ROWS // RB
BIG_W = FULL_BLOCKS - 12 * NW

B1, B2, B3, B4, B5, B6, B7, B8, B9 = 0, 40, 80, 100, 120, 140, 160, 180, 220
SLOTS = 272
EPS = np.float32(1e-6)
HALF = np.float32(0.5)
F0 = np.float32(0.0)
F1 = np.float32(1.0)

INV40 = np.float32(1.0) / np.float32(2.0 / 40.0)
INV20 = np.float32(1.0) / np.float32(1.0 / 20.0)
INV20W = np.float32(1.0) / np.float32(2.0 / 20.0)
F40 = np.float32(40.0)
F20 = np.float32(20.0)


def _clip_bins(t, nbins):
  i = plsc.bitcast(t.astype(jnp.int32), jnp.uint32)
  return plsc.bitcast(jnp.minimum(i, np.uint32(nbins - 1)), jnp.int32)


def _sc_body(pb_hbm, gt_hbm, mb_hbm, pbt_hbm, gtt_hbm, mbt_hbm, out_hbm,
             pb_v, gt_v, mb_v, part_v, hist):
  wid = lax.axis_index("s") * NC + lax.axis_index("c")
  lane = lax.iota(jnp.int32, L)
  zeros = jnp.zeros((L,), jnp.float32)
  ones = jnp.ones((L,), jnp.float32)

  lh1 = lane + B1 * L
  lh2 = lane + B2 * L
  lh3 = lane + B3 * L
  D8 = (B8 - B1) * L
  D9 = (B9 - B2) * L

  def zero_slot(s, _):
    hist[pl.ds(s * L, L)] = zeros
    return 0

  lax.fori_loop(0, SLOTS, zero_slot, 0)

  block0 = wid * 12 + jnp.minimum(wid, BIG_W)
  n_blocks = jnp.where(wid < BIG_W, 13, 12)

  def do_block(nsub):
    def sub_body(k, _):
      k16 = k * L
      gx1 = gt_v[0, 0, pl.ds(k16, L)]
      gy1 = gt_v[0, 1, pl.ds(k16, L)]
      gx2 = gt_v[0, 2, pl.ds(k16, L)]
      gy2 = gt_v[0, 3, pl.ds(k16, L)]
      mx1 = mb_v[0, 0, pl.ds(k16, L)]
      my1 = mb_v[0, 1, pl.ds(k16, L)]
      mx2 = mb_v[0, 2, pl.ds(k16, L)]
      my2 = mb_v[0, 3, pl.ds(k16, L)]

      area_g = (gx2 - gx1) * (gy2 - gy1)
      area_m = (mx2 - mx1) * (my2 - my1)
      ww = jnp.maximum(jnp.minimum(gx2, mx2) - jnp.maximum(gx1, mx1), F0)
      hh = jnp.maximum(jnp.minimum(gy2, my2) - jnp.maximum(gy1, my1), F0)
      ov = ww * hh
      giou = ov / jnp.maximum(area_g + area_m - ov, EPS)
      recip_g = F1 / giou
      rn = recip_g * INV40
      a_m = (F1 - giou) * INV40
      g_big = giou >= HALF

      @plsc.parallel_loop(0, N_BOX, unroll=2,
                          carry=jnp.full((L,), -jnp.inf, jnp.float32))
      def max_iou1(j, run_max):
        px1 = pb_v[j, 0, pl.ds(k16, L)]
        py1 = pb_v[j, 1, pl.ds(k16, L)]
        px2 = pb_v[j, 2, pl.ds(k16, L)]
        py2 = pb_v[j, 3, pl.ds(k16, L)]
        area_p = (px2 - px1) * (py2 - py1)
        iw = jnp.maximum(jnp.minimum(px2, gx2) - jnp.maximum(px1, gx1), F0)
        ih = jnp.maximum(jnp.minimum(py2, gy2) - jnp.maximum(py1, gy1), F0)
        iov = iw * ih
        iou1 = iov / jnp.maximum(area_p + area_g - iov, EPS)

        t_n = iou1 * rn
        t_m = iou1 * INV40 + a_m
        t_1 = iou1 * INV20
        i_n = _clip_bins(t_n, 40)
        i_m = _clip_bins(t_m, 40)
        i_1 = _clip_bins(t_1, 20)
        v_n = t_n <= F40
        v_m = (t_m >= F0) & (t_m <= F40)
        v_1 = t_1 <= F20
        nb = (iou1 >= HALF) | g_big

        a_n = i_n * L + lh1
        a_m2 = i_m * L + lh2
        a_1 = i_1 * L + lh3
        plsc.addupdate_scatter(hist, [a_n], ones, mask=v_n)
        plsc.addupdate_scatter(hist, [a_m2], ones, mask=v_m)
        plsc.addupdate_scatter(hist, [a_1], ones, mask=v_1)
        plsc.addupdate_scatter(hist, [a_n + D8], ones, mask=v_n & nb)
        plsc.addupdate_scatter(hist, [a_m2 + D9], ones, mask=v_m & nb)
        return jnp.maximum(run_max, iou1)

      t4 = giou * INV20
      i4 = _clip_bins(t4, 20)
      v4 = t4 <= F20
      t5 = max_iou1 * INV20
      i5 = _clip_bins(t5, 20)
      v5 = (t5 >= F0) & (t5 <= F20)
      t7 = (giou + F1) * INV20W
      i7 = _clip_bins(t7, 20)
      v7 = t7 <= F20
      plsc.addupdate_scatter(hist, [i4 * L + (lane + B4 * L)],
                             jnp.full((L,), 64.0, jnp.float32), mask=v4)
      plsc.addupdate_scatter(hist, [i5 * L + (lane + B5 * L)], ones, mask=v5)
      plsc.addupdate_scatter(hist, [i4 * L + (lane + B6 * L)], ones, mask=v4)
      plsc.addupdate_scatter(hist, [i7 * L + (lane + B7 * L)], ones, mask=v7)
      return 0

    lax.fori_loop(0, nsub, sub_body, 0)

  def block_body(s, _):
    r0 = (block0 + s) * RB
    pltpu.sync_copy(pb_hbm.at[:, :, pl.ds(r0, RB)], pb_v)
    pltpu.sync_copy(gt_hbm.at[:, :, pl.ds(r0, RB)], gt_v)
    pltpu.sync_copy(mb_hbm.at[:, :, pl.ds(r0, RB)], mb_v)
    do_block(RB // L)
    return 0

  lax.fori_loop(0, n_blocks, block_body, 0)

  @pl.when(wid == NW - 1)
  def _():
    pltpu.sync_copy(pbt_hbm, pb_v)
    pltpu.sync_copy(gtt_hbm, gt_v)
    pltpu.sync_copy(mbt_hbm, mb_v)
    do_block(RB // L)

  for cc in range(SLOTS // L):
    base = (cc * L + lane) * L
    acc = zeros
    for l in range(L):
      acc = acc + plsc.load_gather(hist, [base + l])
    part_v[0, pl.ds(cc * L, L)] = acc

  pltpu.sync_copy(part_v, out_hbm.at[pl.ds(wid, 1)])


@jax.jit
def kernel(pseudo_boxes, gt_bboxes, merge_boxes):
  pb = jnp.transpose(pseudo_boxes, (1, 2, 0))
  gt = jnp.transpose(gt_bboxes, (1, 2, 0))
  mb = jnp.transpose(merge_boxes, (1, 2, 0))
  r0 = FULL_BLOCKS * RB
  padw = ((0, 0), (0, 0), (0, RB - (N_ROWS - r0)))
  nan = np.float32(np.nan)
  pbt = jnp.pad(pb[:, :, r0:], padw, constant_values=nan)
  gtt = jnp.pad(gt[:, :, r0:], padw, constant_values=nan)
  mbt = jnp.pad(mb[:, :, r0:], padw, constant_values=nan)
  mesh = plsc.VectorSubcoreMesh(core_axis_name="c", subcore_axis_name="s")
  run = pl.kernel(
      _sc_body,
      out_type=jax.ShapeDtypeStruct((NW, SLOTS), jnp.float32),
      mesh=mesh,
      compiler_params=pltpu.CompilerParams(needs_layout_passes=False),
      scratch_types=[
          pltpu.VMEM((N_BOX, 4, RB), jnp.float32),
          pltpu.VMEM((1, 4, RB), jnp.float32),
          pltpu.VMEM((1, 4, RB), jnp.float32),
          pltpu.VMEM((1, SLOTS), jnp.float32),
          pltpu.VMEM((SLOTS * L, ), jnp.float32),
      ],
  )
  partials = run(pb, gt, mb, pbt, gtt, mbt)
  return jnp.sum(partials, axis=0)[:260]
